# Initial kernel scaffold; baseline (speedup 1.0000x reference)
import functools

import jax
import jax.numpy as jnp
from jax.experimental import pallas as pl
from jax.experimental.pallas import tpu as pltpu

T, D, E, F = 2048, 768, 8, 1536
BT = 1024


def _combined_weights(logits):
    p = jax.nn.softmax(logits, axis=-1)
    e_iota = jax.lax.broadcasted_iota(jnp.int32, p.shape, 1)
    m1 = jnp.max(p, axis=-1, keepdims=True)
    i1 = jnp.argmax(p, axis=-1)[:, None]
    p2 = jnp.where(e_iota == i1, -jnp.inf, p)
    m2 = jnp.max(p2, axis=-1, keepdims=True)
    i2 = jnp.argmax(p2, axis=-1)[:, None]
    w1 = 1.0 / (1.0 + jnp.exp(m2 - m1))
    w2 = 1.0 - w1
    c = jnp.where(e_iota == i1, w1, 0.0) + jnp.where(e_iota == i2, w2, 0.0)
    return c


def _moe_body(x_ref, rw_ref, rb_ref, w1_ref, b1_ref, w2_ref, b2_ref,
              out_ref, c_ref):
    e = pl.program_id(1)

    @pl.when(e == 0)
    def _():
        logits = jnp.dot(x_ref[...], rw_ref[...],
                         preferred_element_type=jnp.float32) + rb_ref[...]
        c_ref[...] = _combined_weights(logits)
        out_ref[...] = jnp.zeros_like(out_ref)

    xb = x_ref[...]
    h = jnp.dot(xb, w1_ref[0], preferred_element_type=jnp.float32) + b1_ref[0]
    h = jax.nn.gelu(h, approximate=False)
    y = jnp.dot(h, w2_ref[0], preferred_element_type=jnp.float32) + b2_ref[0]
    ce = c_ref[:, e][:, None]
    out_ref[...] += ce * y


@jax.jit
def kernel(x, router_w, router_b, w1, b1, w2, b2):
    rb = router_b.reshape(1, E)
    b1r = b1.reshape(E, 1, F)
    b2r = b2.reshape(E, 1, D)
    grid = (T // BT, E)
    out = pl.pallas_call(
        _moe_body,
        grid=grid,
        in_specs=[
            pl.BlockSpec((BT, D), lambda t, e: (t, 0)),
            pl.BlockSpec((D, E), lambda t, e: (0, 0)),
            pl.BlockSpec((1, E), lambda t, e: (0, 0)),
            pl.BlockSpec((1, D, F), lambda t, e: (e, 0, 0)),
            pl.BlockSpec((1, 1, F), lambda t, e: (e, 0, 0)),
            pl.BlockSpec((1, F, D), lambda t, e: (e, 0, 0)),
            pl.BlockSpec((1, 1, D), lambda t, e: (e, 0, 0)),
        ],
        out_specs=pl.BlockSpec((BT, D), lambda t, e: (t, 0)),
        out_shape=jax.ShapeDtypeStruct((T, D), jnp.float32),
        scratch_shapes=[pltpu.VMEM((BT, E), jnp.float32)],
    )(x, router_w, rb, w1, b1r, w2, b2r)
    return out


# speedup vs baseline: 2.2358x; 2.2358x over previous
import functools

import jax
import jax.numpy as jnp
from jax.experimental import pallas as pl
from jax.experimental.pallas import tpu as pltpu

T, D, E, F = 2048, 768, 8, 1536
BT = 1024


def _erf(x):
    s = jnp.sign(x)
    a = jnp.abs(x)
    t = 1.0 / (1.0 + 0.3275911 * a)
    poly = t * (0.254829592 + t * (-0.284496736 + t * (1.421413741
              + t * (-1.453152027 + t * 1.061405429))))
    return s * (1.0 - poly * jnp.exp(-a * a))


def _gelu_exact(x):
    return 0.5 * x * (1.0 + _erf(x * 0.7071067811865476))


def _combined_weights(logits):
    p = jax.nn.softmax(logits, axis=-1)
    e_iota = jax.lax.broadcasted_iota(jnp.int32, p.shape, 1)
    m1 = jnp.max(p, axis=-1, keepdims=True)
    i1 = jnp.argmax(p, axis=-1)[:, None]
    p2 = jnp.where(e_iota == i1, -jnp.inf, p)
    m2 = jnp.max(p2, axis=-1, keepdims=True)
    i2 = jnp.argmax(p2, axis=-1)[:, None]
    w1 = 1.0 / (1.0 + jnp.exp(m2 - m1))
    w2 = 1.0 - w1
    c = jnp.where(e_iota == i1, w1, 0.0) + jnp.where(e_iota == i2, w2, 0.0)
    return c


def _moe_body(x_ref, rw_ref, rb_ref, w1_ref, b1_ref, w2_ref, b2_ref,
              out_ref, c_ref):
    e = pl.program_id(1)

    @pl.when(e == 0)
    def _():
        logits = jnp.dot(x_ref[...], rw_ref[...],
                         preferred_element_type=jnp.float32) + rb_ref[...]
        c_ref[...] = _combined_weights(logits)
        out_ref[...] = jnp.zeros_like(out_ref)

    xb = x_ref[...]
    h = jnp.dot(xb, w1_ref[0], preferred_element_type=jnp.float32) + b1_ref[0]
    h = _gelu_exact(h)
    y = jnp.dot(h, w2_ref[0], preferred_element_type=jnp.float32) + b2_ref[0]
    c = c_ref[...]
    lane = jax.lax.broadcasted_iota(jnp.int32, c.shape, 1)
    ce = jnp.sum(jnp.where(lane == e, c, 0.0), axis=1, keepdims=True)
    out_ref[...] += ce * y


@jax.jit
def kernel(x, router_w, router_b, w1, b1, w2, b2):
    rb = router_b.reshape(1, E)
    b1r = b1.reshape(E, 1, F)
    b2r = b2.reshape(E, 1, D)
    grid = (T // BT, E)
    out = pl.pallas_call(
        _moe_body,
        grid=grid,
        in_specs=[
            pl.BlockSpec((BT, D), lambda t, e: (t, 0)),
            pl.BlockSpec((D, E), lambda t, e: (0, 0)),
            pl.BlockSpec((1, E), lambda t, e: (0, 0)),
            pl.BlockSpec((1, D, F), lambda t, e: (e, 0, 0)),
            pl.BlockSpec((1, 1, F), lambda t, e: (e, 0, 0)),
            pl.BlockSpec((1, F, D), lambda t, e: (e, 0, 0)),
            pl.BlockSpec((1, 1, D), lambda t, e: (e, 0, 0)),
        ],
        out_specs=pl.BlockSpec((BT, D), lambda t, e: (t, 0)),
        out_shape=jax.ShapeDtypeStruct((T, D), jnp.float32),
        scratch_shapes=[pltpu.VMEM((BT, E), jnp.float32)],
    )(x, router_w, rb, w1, b1r, w2, b2r)
    return out
